# baseline (speedup 1.0000x reference)
import jax
import jax.numpy as jnp
from jax.experimental import pallas as pl
from jax.experimental.pallas import tpu as pltpu

D_MODEL_ = 512
D_STATE_ = 16
D_CONV_ = 4
D_INNER_ = 1024
DT_RANK_ = 32

_RT1 = 512
_CT1 = 1024
_TC = 256
_GRP = 8
_RT3 = 512


def _ln_inproj_kernel(x_ref, j_ref, g_ref, b_ref, w_ref, o_ref):
    d = pl.program_id(0)
    xt = x_ref[0]
    mu = jnp.mean(xt, axis=1, keepdims=True)
    xc = xt - mu
    var = jnp.mean(xc * xc, axis=1, keepdims=True)
    xn = xc * jax.lax.rsqrt(var + 1e-5) * g_ref[...] + b_ref[...]
    xn = jnp.where(d == 0, xn,
                   jnp.dot(j_ref[...], xn, preferred_element_type=jnp.float32))
    o_ref[0, 0] = jnp.dot(xn, w_ref[0], preferred_element_type=jnp.float32)


def _scan_kernel(x_ref, xh_ref, jt_ref, j8_ref, g_ref, b_ref, win_ref,
                 convw_ref, convb_ref, xpdt_ref, dtw_ref,
                 dtb_ref, bcw_ref, at_ref, dp_ref, y_ref,
                 z_s, xc_s, dt_s, bct_s, h_s):
    d = pl.program_id(0)
    c = pl.program_id(2)
    tc = xc_s.shape[0]

    def ln(t):
        mu = jnp.mean(t, axis=1, keepdims=True)
        v = t - mu
        var = jnp.mean(v * v, axis=1, keepdims=True)
        return v * jax.lax.rsqrt(var + 1e-5) * g_ref[...] + b_ref[...]

    xn = ln(x_ref[0])
    xn = jnp.where(d == 0, xn,
                   jnp.dot(jt_ref[...], xn, preferred_element_type=jnp.float32))
    xnh = ln(xh_ref[0])
    xnh = jnp.where(d == 0, xnh,
                    jnp.dot(j8_ref[...], xnh,
                            preferred_element_type=jnp.float32))
    xzc = jnp.dot(xn, win_ref[0], preferred_element_type=jnp.float32)
    xi = xzc[:, :D_INNER_]
    z_s[...] = xzc[:, D_INNER_:]
    xih = jnp.dot(xnh, win_ref[0], preferred_element_type=jnp.float32)

    prev3 = xih[5:8, :D_INNER_]
    prev3 = jnp.where(c == 0, 0.0, prev3)
    xp = jnp.concatenate([prev3, xi], axis=0)
    w = convw_ref[0]
    xc = (xp[0:tc] * w[0:1, :] + xp[1:tc + 1] * w[1:2, :]
          + xp[2:tc + 2] * w[2:3, :] + xp[3:tc + 3] * w[3:4, :]
          + convb_ref[0])
    xc = xc * jax.nn.sigmoid(xc)
    xc_s[...] = xc

    dtr = jnp.dot(xc, xpdt_ref[0], preferred_element_type=jnp.float32)
    dt_lin = jnp.dot(dtr, dtw_ref[0],
                     preferred_element_type=jnp.float32) + dtb_ref[0]
    dt_s[...] = jax.nn.softplus(dt_lin)
    bct_s[...] = jax.lax.dot_general(
        bcw_ref[0], xc, (((1,), (1,)), ((), ())),
        preferred_element_type=jnp.float32)

    aT = at_ref[0]

    @pl.when(c == 0)
    def _():
        h_s[...] = jnp.zeros_like(h_s)

    bct_all = bct_s[...]

    def subgroup(base, h):
        dt8 = dt_s[pl.ds(base, _GRP), :]
        xc8 = xc_s[pl.ds(base, _GRP), :]
        bc8 = pltpu.roll(bct_all, -base, axis=1)[:, :_GRP]
        dA8 = jnp.exp(dt8[:, None, :] * aT[None])
        bx8 = (dt8 * xc8)[:, None, :]
        ys = []
        for r in range(_GRP):
            bcol = bc8[0:D_STATE_, r:r + 1]
            ccol = bc8[D_STATE_:2 * D_STATE_, r:r + 1]
            h = dA8[r] * h + bx8[r] * bcol
            ys.append(jnp.sum(ccol * h, axis=0, keepdims=True))
        y_ref[0, 0, pl.ds(base, _GRP), :] = jnp.concatenate(ys, axis=0)
        return h

    def body(j, carry):
        base = j * (8 * _GRP)
        h = h_s[...]
        for s in range(8):
            h = subgroup(base + s * _GRP, h)
        h_s[...] = h
        return carry

    jax.lax.fori_loop(0, tc // (8 * _GRP), body, 0)

    z = z_s[...]
    y_ref[0, 0] = ((y_ref[0, 0] + xc_s[...] * dp_ref[0])
                   * (z * jax.nn.sigmoid(z)))


def _merge_kernel(yf_ref, yb_ref, j_ref, wf_ref, wb_ref, x_ref, mb_ref, o_ref):
    ub = jnp.dot(yb_ref[0, 0], wb_ref[...], preferred_element_type=jnp.float32)
    o_ref[0] = (jnp.dot(yf_ref[0, 0], wf_ref[...],
                        preferred_element_type=jnp.float32)
                + jnp.dot(j_ref[...], ub, preferred_element_type=jnp.float32)
                + x_ref[0] + mb_ref[...])


def kernel(x, norm_g, norm_b, p_fwd, p_bwd, merge_w, merge_b):
    B, L, dm = x.shape
    di = D_INNER_
    ds = D_STATE_

    def prep(p):
        in_w, conv_w, conv_b, xproj_w, dt_w, dt_b, A_log, Dp, out_w = p
        return dict(
            winT=in_w.T,
            convw=conv_w.T,
            convb=conv_b[None, :],
            xpdtT=xproj_w[:DT_RANK_].T,
            bcw=xproj_w[DT_RANK_:],
            dtwT=dt_w.T,
            dtb=dt_b[None, :],
            aT=(-jnp.exp(A_log)).T,
            dp=Dp[None, :],
            out_w=out_w,
        )

    pf, pb = prep(p_fwd), prep(p_bwd)

    def stk(k):
        return jnp.stack([pf[k], pb[k]])

    winT = stk('winT')
    convw = stk('convw')
    convb = stk('convb')
    xpdtT = stk('xpdtT')
    bcw = stk('bcw')
    dtwT = stk('dtwT')
    dtb = stk('dtb')
    aT = stk('aT')
    dp = stk('dp')

    hp = jax.lax.Precision.HIGHEST
    wfT = jnp.dot(pf['out_w'].T, merge_w[:, :dm].T, precision=hp)
    wbT = jnp.dot(pb['out_w'].T, merge_w[:, dm:].T, precision=hp)

    nc = L // _TC
    nbr = L // 8
    jrevt = jnp.flipud(jnp.eye(_TC, dtype=jnp.float32))
    jrev8 = jnp.flipud(jnp.eye(8, dtype=jnp.float32))
    y = pl.pallas_call(
        _scan_kernel,
        grid=(2, B, nc),
        in_specs=[
            pl.BlockSpec((1, _TC, dm),
                         lambda d, b, c: (b, c + d * (nc - 1 - 2 * c), 0)),
            pl.BlockSpec((1, 8, dm),
                         lambda d, b, c: (b, jnp.where(
                             d == 0,
                             jnp.maximum(c * (_TC // 8) - 1, 0),
                             jnp.minimum((nc - c) * (_TC // 8), nbr - 1)), 0)),
            pl.BlockSpec((_TC, _TC), lambda d, b, c: (0, 0)),
            pl.BlockSpec((8, 8), lambda d, b, c: (0, 0)),
            pl.BlockSpec((1, dm), lambda d, b, c: (0, 0)),
            pl.BlockSpec((1, dm), lambda d, b, c: (0, 0)),
            pl.BlockSpec((1, dm, 2 * di), lambda d, b, c: (d, 0, 0)),
            pl.BlockSpec((1, 4, di), lambda d, b, c: (d, 0, 0)),
            pl.BlockSpec((1, 1, di), lambda d, b, c: (d, 0, 0)),
            pl.BlockSpec((1, di, DT_RANK_), lambda d, b, c: (d, 0, 0)),
            pl.BlockSpec((1, DT_RANK_, di), lambda d, b, c: (d, 0, 0)),
            pl.BlockSpec((1, 1, di), lambda d, b, c: (d, 0, 0)),
            pl.BlockSpec((1, 2 * ds, di), lambda d, b, c: (d, 0, 0)),
            pl.BlockSpec((1, ds, di), lambda d, b, c: (d, 0, 0)),
            pl.BlockSpec((1, 1, di), lambda d, b, c: (d, 0, 0)),
        ],
        out_specs=pl.BlockSpec((1, 1, _TC, di), lambda d, b, c: (d, b, c, 0)),
        out_shape=jax.ShapeDtypeStruct((2, B, L, di), jnp.float32),
        scratch_shapes=[
            pltpu.VMEM((_TC, di), jnp.float32),
            pltpu.VMEM((_TC, di), jnp.float32),
            pltpu.VMEM((_TC, di), jnp.float32),
            pltpu.VMEM((2 * ds, _TC), jnp.float32),
            pltpu.VMEM((ds, di), jnp.float32),
        ],
        compiler_params=pltpu.CompilerParams(
            dimension_semantics=("parallel", "parallel", "arbitrary"),
            vmem_limit_bytes=50 * 1024 * 1024),
        name="mamba_scan",
    )(x, x, jrevt, jrev8, norm_g[None, :], norm_b[None, :], winT,
      convw, convb, xpdtT, dtwT, dtb, bcw, aT, dp)

    nrt3 = L // _RT3
    jrev3 = jnp.flipud(jnp.eye(_RT3, dtype=jnp.float32))
    out = pl.pallas_call(
        _merge_kernel,
        grid=(B, nrt3),
        in_specs=[
            pl.BlockSpec((1, 1, _RT3, di), lambda b, r: (0, b, r, 0)),
            pl.BlockSpec((1, 1, _RT3, di), lambda b, r: (1, b, nrt3 - 1 - r, 0)),
            pl.BlockSpec((_RT3, _RT3), lambda b, r: (0, 0)),
            pl.BlockSpec((di, dm), lambda b, r: (0, 0)),
            pl.BlockSpec((di, dm), lambda b, r: (0, 0)),
            pl.BlockSpec((1, _RT3, dm), lambda b, r: (b, r, 0)),
            pl.BlockSpec((1, dm), lambda b, r: (0, 0)),
        ],
        out_specs=pl.BlockSpec((1, _RT3, dm), lambda b, r: (b, r, 0)),
        out_shape=jax.ShapeDtypeStruct((B, L, dm), jnp.float32),
        compiler_params=pltpu.CompilerParams(
            dimension_semantics=("parallel", "parallel")),
        name="merge",
    )(y, y, jrev3, wfT, wbT, x, merge_b[None, :])

    return out


# speedup vs baseline: 1.0923x; 1.0923x over previous
import jax
import jax.numpy as jnp
from jax.experimental import pallas as pl
from jax.experimental.pallas import tpu as pltpu

D_MODEL_ = 512
D_STATE_ = 16
D_CONV_ = 4
D_INNER_ = 1024
DT_RANK_ = 32

_TC = 256
_GRP = 8
_RT3 = 512


def _scan_kernel(x_ref, xh_ref, jt_ref, j8_ref, g_ref, b_ref, win_ref,
                 convw_ref, convb_ref, xpdt_ref, dtw_ref,
                 dtb_ref, bcw_ref, at_ref, dp_ref, y_ref,
                 z_s, xc_s, dt_s, bct_s, h_s):
    d = pl.program_id(0)
    c = pl.program_id(2)
    tc = xc_s.shape[0]

    def ln(t):
        mu = jnp.mean(t, axis=1, keepdims=True)
        v = t - mu
        var = jnp.mean(v * v, axis=1, keepdims=True)
        return v * jax.lax.rsqrt(var + 1e-5) * g_ref[...] + b_ref[...]

    xn = ln(x_ref[0])
    xn = jnp.where(d == 0, xn,
                   jnp.dot(jt_ref[...], xn, preferred_element_type=jnp.float32))
    xnh = ln(xh_ref[0])
    xnh = jnp.where(d == 0, xnh,
                    jnp.dot(j8_ref[...], xnh,
                            preferred_element_type=jnp.float32))
    xzc = jnp.dot(xn, win_ref[0], preferred_element_type=jnp.float32)
    xi = xzc[:, :D_INNER_]
    z_s[...] = xzc[:, D_INNER_:]
    xih = jnp.dot(xnh, win_ref[0], preferred_element_type=jnp.float32)

    prev3 = xih[5:8, :D_INNER_]
    prev3 = jnp.where(c == 0, 0.0, prev3)
    xp = jnp.concatenate([prev3, xi], axis=0)
    w = convw_ref[0]
    xc = (xp[0:tc] * w[0:1, :] + xp[1:tc + 1] * w[1:2, :]
          + xp[2:tc + 2] * w[2:3, :] + xp[3:tc + 3] * w[3:4, :]
          + convb_ref[0])
    xc = xc * jax.nn.sigmoid(xc)
    xc_s[...] = xc

    dtr = jnp.dot(xc, xpdt_ref[0], preferred_element_type=jnp.float32)
    dt_lin = jnp.dot(dtr, dtw_ref[0],
                     preferred_element_type=jnp.float32) + dtb_ref[0]
    dt_s[...] = jax.nn.softplus(dt_lin)
    bct_s[...] = jax.lax.dot_general(
        bcw_ref[0], xc, (((1,), (1,)), ((), ())),
        preferred_element_type=jnp.float32)

    aT = at_ref[0]

    @pl.when(c == 0)
    def _():
        h_s[...] = jnp.zeros_like(h_s)

    bct_all = bct_s[...]

    def subgroup(base, h):
        dt8 = dt_s[pl.ds(base, _GRP), :]
        xc8 = xc_s[pl.ds(base, _GRP), :]
        bc8 = pltpu.roll(bct_all, -base, axis=1)[:, :_GRP]
        dA8 = jnp.exp(dt8[:, None, :] * aT[None])
        bx8 = (dt8 * xc8)[:, None, :]
        ys = []
        for r in range(_GRP):
            bcol = bc8[0:D_STATE_, r:r + 1]
            ccol = bc8[D_STATE_:2 * D_STATE_, r:r + 1]
            h = dA8[r] * h + bx8[r] * bcol
            ys.append(jnp.sum(ccol * h, axis=0, keepdims=True))
        y_ref[0, 0, pl.ds(base, _GRP), :] = jnp.concatenate(ys, axis=0)
        return h

    def body(j, carry):
        base = j * (4 * _GRP)
        h = h_s[...]
        for s in range(4):
            h = subgroup(base + s * _GRP, h)
        h_s[...] = h
        return carry

    jax.lax.fori_loop(0, tc // (4 * _GRP), body, 0)

    z = z_s[...]
    y_ref[0, 0] = ((y_ref[0, 0] + xc_s[...] * dp_ref[0])
                   * (z * jax.nn.sigmoid(z)))


def _merge_kernel(yf_ref, yb_ref, j_ref, wf_ref, wb_ref, x_ref, mb_ref, o_ref):
    ub = jnp.dot(yb_ref[0, 0], wb_ref[...], preferred_element_type=jnp.float32)
    o_ref[0] = (jnp.dot(yf_ref[0, 0], wf_ref[...],
                        preferred_element_type=jnp.float32)
                + jnp.dot(j_ref[...], ub, preferred_element_type=jnp.float32)
                + x_ref[0] + mb_ref[...])


def kernel(x, norm_g, norm_b, p_fwd, p_bwd, merge_w, merge_b):
    B, L, dm = x.shape
    di = D_INNER_
    ds = D_STATE_

    def prep(p):
        in_w, conv_w, conv_b, xproj_w, dt_w, dt_b, A_log, Dp, out_w = p
        return dict(
            winT=in_w.T,
            convw=conv_w.T,
            convb=conv_b[None, :],
            xpdtT=xproj_w[:DT_RANK_].T,
            bcw=xproj_w[DT_RANK_:],
            dtwT=dt_w.T,
            dtb=dt_b[None, :],
            aT=(-jnp.exp(A_log)).T,
            dp=Dp[None, :],
            out_w=out_w,
        )

    pf, pb = prep(p_fwd), prep(p_bwd)

    def stk(k):
        return jnp.stack([pf[k], pb[k]])

    winT = stk('winT')
    convw = stk('convw')
    convb = stk('convb')
    xpdtT = stk('xpdtT')
    bcw = stk('bcw')
    dtwT = stk('dtwT')
    dtb = stk('dtb')
    aT = stk('aT')
    dp = stk('dp')

    hp = jax.lax.Precision.HIGHEST
    wfT = jnp.dot(pf['out_w'].T, merge_w[:, :dm].T, precision=hp)
    wbT = jnp.dot(pb['out_w'].T, merge_w[:, dm:].T, precision=hp)

    nc = L // _TC
    nbr = L // 8
    jrevt = jnp.flipud(jnp.eye(_TC, dtype=jnp.float32))
    jrev8 = jnp.flipud(jnp.eye(8, dtype=jnp.float32))
    y = pl.pallas_call(
        _scan_kernel,
        grid=(2, B, nc),
        in_specs=[
            pl.BlockSpec((1, _TC, dm),
                         lambda d, b, c: (b, c + d * (nc - 1 - 2 * c), 0)),
            pl.BlockSpec((1, 8, dm),
                         lambda d, b, c: (b, jnp.where(
                             d == 0,
                             jnp.maximum(c * (_TC // 8) - 1, 0),
                             jnp.minimum((nc - c) * (_TC // 8), nbr - 1)), 0)),
            pl.BlockSpec((_TC, _TC), lambda d, b, c: (0, 0)),
            pl.BlockSpec((8, 8), lambda d, b, c: (0, 0)),
            pl.BlockSpec((1, dm), lambda d, b, c: (0, 0)),
            pl.BlockSpec((1, dm), lambda d, b, c: (0, 0)),
            pl.BlockSpec((1, dm, 2 * di), lambda d, b, c: (d, 0, 0)),
            pl.BlockSpec((1, 4, di), lambda d, b, c: (d, 0, 0)),
            pl.BlockSpec((1, 1, di), lambda d, b, c: (d, 0, 0)),
            pl.BlockSpec((1, di, DT_RANK_), lambda d, b, c: (d, 0, 0)),
            pl.BlockSpec((1, DT_RANK_, di), lambda d, b, c: (d, 0, 0)),
            pl.BlockSpec((1, 1, di), lambda d, b, c: (d, 0, 0)),
            pl.BlockSpec((1, 2 * ds, di), lambda d, b, c: (d, 0, 0)),
            pl.BlockSpec((1, ds, di), lambda d, b, c: (d, 0, 0)),
            pl.BlockSpec((1, 1, di), lambda d, b, c: (d, 0, 0)),
        ],
        out_specs=pl.BlockSpec((1, 1, _TC, di), lambda d, b, c: (d, b, c, 0)),
        out_shape=jax.ShapeDtypeStruct((2, B, L, di), jnp.float32),
        scratch_shapes=[
            pltpu.VMEM((_TC, di), jnp.float32),
            pltpu.VMEM((_TC, di), jnp.float32),
            pltpu.VMEM((_TC, di), jnp.float32),
            pltpu.VMEM((2 * ds, _TC), jnp.float32),
            pltpu.VMEM((ds, di), jnp.float32),
        ],
        compiler_params=pltpu.CompilerParams(
            dimension_semantics=("parallel", "parallel", "arbitrary"),
            vmem_limit_bytes=50 * 1024 * 1024),
        name="mamba_scan",
    )(x, x, jrevt, jrev8, norm_g[None, :], norm_b[None, :], winT,
      convw, convb, xpdtT, dtwT, dtb, bcw, aT, dp)

    nrt3 = L // _RT3
    jrev3 = jnp.flipud(jnp.eye(_RT3, dtype=jnp.float32))
    out = pl.pallas_call(
        _merge_kernel,
        grid=(B, nrt3),
        in_specs=[
            pl.BlockSpec((1, 1, _RT3, di), lambda b, r: (0, b, r, 0)),
            pl.BlockSpec((1, 1, _RT3, di), lambda b, r: (1, b, nrt3 - 1 - r, 0)),
            pl.BlockSpec((_RT3, _RT3), lambda b, r: (0, 0)),
            pl.BlockSpec((di, dm), lambda b, r: (0, 0)),
            pl.BlockSpec((di, dm), lambda b, r: (0, 0)),
            pl.BlockSpec((1, _RT3, dm), lambda b, r: (b, r, 0)),
            pl.BlockSpec((1, dm), lambda b, r: (0, 0)),
        ],
        out_specs=pl.BlockSpec((1, _RT3, dm), lambda b, r: (b, r, 0)),
        out_shape=jax.ShapeDtypeStruct((B, L, dm), jnp.float32),
        compiler_params=pltpu.CompilerParams(
            dimension_semantics=("parallel", "parallel")),
        name="merge",
    )(y, y, jrev3, wfT, wbT, x, merge_b[None, :])

    return out


# speedup vs baseline: 1.1241x; 1.0291x over previous
import jax
import jax.numpy as jnp
from jax.experimental import pallas as pl
from jax.experimental.pallas import tpu as pltpu

D_MODEL_ = 512
D_STATE_ = 16
D_CONV_ = 4
D_INNER_ = 1024
DT_RANK_ = 32

_TC = 256
_GRP = 8
_RT3 = 512


def _scan_kernel(x_ref, xh_ref, jt_ref, j8_ref, g_ref, b_ref, win_ref,
                 convw_ref, convb_ref, xpdt_ref, dtw_ref,
                 dtb_ref, bcw_ref, at_ref, dp_ref, y_ref,
                 z_s, xc_s, dt_s, bct_s, h_s):
    d = pl.program_id(0)
    c = pl.program_id(2)
    tc = xc_s.shape[0]

    def ln(t):
        mu = jnp.mean(t, axis=1, keepdims=True)
        v = t - mu
        var = jnp.mean(v * v, axis=1, keepdims=True)
        return v * jax.lax.rsqrt(var + 1e-5) * g_ref[...] + b_ref[...]

    xn = ln(x_ref[0])
    xn = jnp.where(d == 0, xn,
                   jnp.dot(jt_ref[...], xn, preferred_element_type=jnp.float32))
    xnh = ln(xh_ref[0])
    xnh = jnp.where(d == 0, xnh,
                    jnp.dot(j8_ref[...], xnh,
                            preferred_element_type=jnp.float32))
    xzc = jnp.dot(xn, win_ref[0], preferred_element_type=jnp.float32)
    xi = xzc[:, :D_INNER_]
    z_s[...] = xzc[:, D_INNER_:]
    xih = jnp.dot(xnh, win_ref[0], preferred_element_type=jnp.float32)

    prev3 = xih[5:8, :D_INNER_]
    prev3 = jnp.where(c == 0, 0.0, prev3)
    xp = jnp.concatenate([prev3, xi], axis=0)
    w = convw_ref[0]
    xc = (xp[0:tc] * w[0:1, :] + xp[1:tc + 1] * w[1:2, :]
          + xp[2:tc + 2] * w[2:3, :] + xp[3:tc + 3] * w[3:4, :]
          + convb_ref[0])
    xc = xc * jax.nn.sigmoid(xc)
    xc_s[...] = xc

    dtr = jnp.dot(xc, xpdt_ref[0], preferred_element_type=jnp.float32)
    dt_lin = jnp.dot(dtr, dtw_ref[0],
                     preferred_element_type=jnp.float32) + dtb_ref[0]
    dt_s[...] = jax.nn.softplus(dt_lin)
    bct_s[...] = jax.lax.dot_general(
        bcw_ref[0], xc, (((1,), (1,)), ((), ())),
        preferred_element_type=jnp.float32)

    aT = at_ref[0]

    @pl.when(c == 0)
    def _():
        h_s[...] = jnp.zeros_like(h_s)

    bct_all = bct_s[...]

    def subgroup(base, dt8, xc8, bc8, h):
        dA8 = jnp.exp(dt8[:, None, :] * aT[None])
        bx8 = (dt8 * xc8)[:, None, :]
        ys = []
        for r in range(_GRP):
            bcol = bc8[0:D_STATE_, r:r + 1]
            ccol = bc8[D_STATE_:2 * D_STATE_, r:r + 1]
            h = dA8[r] * h + bx8[r] * bcol
            ys.append(jnp.sum(ccol * h, axis=0, keepdims=True))
        y_ref[0, 0, pl.ds(base, _GRP), :] = jnp.concatenate(ys, axis=0)
        return h

    nsub = 4
    def body(j, carry):
        base = j * (nsub * _GRP)
        dtg = dt_s[pl.ds(base, nsub * _GRP), :]
        xcg = xc_s[pl.ds(base, nsub * _GRP), :]
        bcg = pltpu.roll(bct_all, -base, axis=1)[:, :nsub * _GRP]
        h = h_s[...]
        for s in range(nsub):
            sl = slice(s * _GRP, (s + 1) * _GRP)
            h = subgroup(base + s * _GRP, dtg[sl], xcg[sl],
                         bcg[:, sl], h)
        h_s[...] = h
        return carry

    jax.lax.fori_loop(0, tc // (nsub * _GRP), body, 0)

    z = z_s[...]
    y_ref[0, 0] = ((y_ref[0, 0] + xc_s[...] * dp_ref[0])
                   * (z * jax.nn.sigmoid(z)))


def _merge_kernel(yf_ref, yb_ref, j_ref, wf_ref, wb_ref, x_ref, mb_ref, o_ref):
    ub = jnp.dot(yb_ref[0, 0], wb_ref[...], preferred_element_type=jnp.float32)
    o_ref[0] = (jnp.dot(yf_ref[0, 0], wf_ref[...],
                        preferred_element_type=jnp.float32)
                + jnp.dot(j_ref[...], ub, preferred_element_type=jnp.float32)
                + x_ref[0] + mb_ref[...])


def kernel(x, norm_g, norm_b, p_fwd, p_bwd, merge_w, merge_b):
    B, L, dm = x.shape
    di = D_INNER_
    ds = D_STATE_

    def prep(p):
        in_w, conv_w, conv_b, xproj_w, dt_w, dt_b, A_log, Dp, out_w = p
        return dict(
            winT=in_w.T,
            convw=conv_w.T,
            convb=conv_b[None, :],
            xpdtT=xproj_w[:DT_RANK_].T,
            bcw=xproj_w[DT_RANK_:],
            dtwT=dt_w.T,
            dtb=dt_b[None, :],
            aT=(-jnp.exp(A_log)).T,
            dp=Dp[None, :],
            out_w=out_w,
        )

    pf, pb = prep(p_fwd), prep(p_bwd)

    def stk(k):
        return jnp.stack([pf[k], pb[k]])

    winT = stk('winT')
    convw = stk('convw')
    convb = stk('convb')
    xpdtT = stk('xpdtT')
    bcw = stk('bcw')
    dtwT = stk('dtwT')
    dtb = stk('dtb')
    aT = stk('aT')
    dp = stk('dp')

    hp = jax.lax.Precision.HIGHEST
    wfT = jnp.dot(pf['out_w'].T, merge_w[:, :dm].T, precision=hp)
    wbT = jnp.dot(pb['out_w'].T, merge_w[:, dm:].T, precision=hp)

    nc = L // _TC
    nbr = L // 8
    jrevt = jnp.flipud(jnp.eye(_TC, dtype=jnp.float32))
    jrev8 = jnp.flipud(jnp.eye(8, dtype=jnp.float32))
    y = pl.pallas_call(
        _scan_kernel,
        grid=(2, B, nc),
        in_specs=[
            pl.BlockSpec((1, _TC, dm),
                         lambda d, b, c: (b, c + d * (nc - 1 - 2 * c), 0)),
            pl.BlockSpec((1, 8, dm),
                         lambda d, b, c: (b, jnp.where(
                             d == 0,
                             jnp.maximum(c * (_TC // 8) - 1, 0),
                             jnp.minimum((nc - c) * (_TC // 8), nbr - 1)), 0)),
            pl.BlockSpec((_TC, _TC), lambda d, b, c: (0, 0)),
            pl.BlockSpec((8, 8), lambda d, b, c: (0, 0)),
            pl.BlockSpec((1, dm), lambda d, b, c: (0, 0)),
            pl.BlockSpec((1, dm), lambda d, b, c: (0, 0)),
            pl.BlockSpec((1, dm, 2 * di), lambda d, b, c: (d, 0, 0)),
            pl.BlockSpec((1, 4, di), lambda d, b, c: (d, 0, 0)),
            pl.BlockSpec((1, 1, di), lambda d, b, c: (d, 0, 0)),
            pl.BlockSpec((1, di, DT_RANK_), lambda d, b, c: (d, 0, 0)),
            pl.BlockSpec((1, DT_RANK_, di), lambda d, b, c: (d, 0, 0)),
            pl.BlockSpec((1, 1, di), lambda d, b, c: (d, 0, 0)),
            pl.BlockSpec((1, 2 * ds, di), lambda d, b, c: (d, 0, 0)),
            pl.BlockSpec((1, ds, di), lambda d, b, c: (d, 0, 0)),
            pl.BlockSpec((1, 1, di), lambda d, b, c: (d, 0, 0)),
        ],
        out_specs=pl.BlockSpec((1, 1, _TC, di), lambda d, b, c: (d, b, c, 0)),
        out_shape=jax.ShapeDtypeStruct((2, B, L, di), jnp.float32),
        scratch_shapes=[
            pltpu.VMEM((_TC, di), jnp.float32),
            pltpu.VMEM((_TC, di), jnp.float32),
            pltpu.VMEM((_TC, di), jnp.float32),
            pltpu.VMEM((2 * ds, _TC), jnp.float32),
            pltpu.VMEM((ds, di), jnp.float32),
        ],
        compiler_params=pltpu.CompilerParams(
            dimension_semantics=("parallel", "parallel", "arbitrary"),
            vmem_limit_bytes=50 * 1024 * 1024),
        name="mamba_scan",
    )(x, x, jrevt, jrev8, norm_g[None, :], norm_b[None, :], winT,
      convw, convb, xpdtT, dtwT, dtb, bcw, aT, dp)

    nrt3 = L // _RT3
    jrev3 = jnp.flipud(jnp.eye(_RT3, dtype=jnp.float32))
    out = pl.pallas_call(
        _merge_kernel,
        grid=(B, nrt3),
        in_specs=[
            pl.BlockSpec((1, 1, _RT3, di), lambda b, r: (0, b, r, 0)),
            pl.BlockSpec((1, 1, _RT3, di), lambda b, r: (1, b, nrt3 - 1 - r, 0)),
            pl.BlockSpec((_RT3, _RT3), lambda b, r: (0, 0)),
            pl.BlockSpec((di, dm), lambda b, r: (0, 0)),
            pl.BlockSpec((di, dm), lambda b, r: (0, 0)),
            pl.BlockSpec((1, _RT3, dm), lambda b, r: (b, r, 0)),
            pl.BlockSpec((1, dm), lambda b, r: (0, 0)),
        ],
        out_specs=pl.BlockSpec((1, _RT3, dm), lambda b, r: (b, r, 0)),
        out_shape=jax.ShapeDtypeStruct((B, L, dm), jnp.float32),
        compiler_params=pltpu.CompilerParams(
            dimension_semantics=("parallel", "parallel")),
        name="merge",
    )(y, y, jrev3, wfT, wbT, x, merge_b[None, :])

    return out


# speedup vs baseline: 1.1511x; 1.0240x over previous
import jax
import jax.numpy as jnp
from jax.experimental import pallas as pl
from jax.experimental.pallas import tpu as pltpu

D_MODEL_ = 512
D_STATE_ = 16
D_CONV_ = 4
D_INNER_ = 1024
DT_RANK_ = 32

_TC = 256
_GRP = 8
_RT3 = 512


def _scan_kernel(x_ref, xh_ref, jt_ref, j8_ref, g_ref, b_ref, win_ref,
                 convw_ref, convb_ref, xpdt_ref, dtw_ref,
                 dtb_ref, bcw_ref, at_ref, dp_ref, y_ref,
                 z_s, xc_s, dt_s, bct_s, h_s):
    d = pl.program_id(0)
    c = pl.program_id(2)
    tc = xc_s.shape[0]

    def ln(t):
        mu = jnp.mean(t, axis=1, keepdims=True)
        v = t - mu
        var = jnp.mean(v * v, axis=1, keepdims=True)
        return v * jax.lax.rsqrt(var + 1e-5) * g_ref[...] + b_ref[...]

    xn = ln(x_ref[0])
    xn = jnp.where(d == 0, xn,
                   jnp.dot(jt_ref[...], xn, preferred_element_type=jnp.float32))
    xnh = ln(xh_ref[0])
    xnh = jnp.where(d == 0, xnh,
                    jnp.dot(j8_ref[...], xnh,
                            preferred_element_type=jnp.float32))
    xzc = jnp.dot(xn, win_ref[0], preferred_element_type=jnp.float32)
    xi = xzc[:, :D_INNER_]
    z_s[...] = xzc[:, D_INNER_:]
    xih = jnp.dot(xnh, win_ref[0], preferred_element_type=jnp.float32)

    prev3 = xih[5:8, :D_INNER_]
    prev3 = jnp.where(c == 0, 0.0, prev3)
    xp = jnp.concatenate([prev3, xi], axis=0)
    w = convw_ref[0]
    xc = (xp[0:tc] * w[0:1, :] + xp[1:tc + 1] * w[1:2, :]
          + xp[2:tc + 2] * w[2:3, :] + xp[3:tc + 3] * w[3:4, :]
          + convb_ref[0])
    xc = xc * jax.nn.sigmoid(xc)
    xc_s[...] = xc

    dtr = jnp.dot(xc, xpdt_ref[0], preferred_element_type=jnp.float32)
    dt_lin = jnp.dot(dtr, dtw_ref[0],
                     preferred_element_type=jnp.float32) + dtb_ref[0]
    dt_s[...] = jax.nn.softplus(dt_lin)
    bct_s[...] = jax.lax.dot_general(
        bcw_ref[0], xc, (((1,), (1,)), ((), ())),
        preferred_element_type=jnp.float32)

    aT = at_ref[0]

    @pl.when(c == 0)
    def _():
        h_s[...] = jnp.zeros_like(h_s)

    bct_all = bct_s[...]

    def subgroup(base, dt8, xc8, bc8, h):
        dA8 = jnp.exp(dt8[:, None, :] * aT[None])
        bx8 = (dt8 * xc8)[:, None, :]
        ys = []
        for r in range(_GRP):
            bcol = bc8[0:D_STATE_, r:r + 1]
            ccol = bc8[D_STATE_:2 * D_STATE_, r:r + 1]
            h = dA8[r] * h + bx8[r] * bcol
            ys.append(jnp.sum(ccol * h, axis=0, keepdims=True))
        y_ref[0, 0, pl.ds(base, _GRP), :] = jnp.concatenate(ys, axis=0)
        return h

    nsub = 8
    def body(j, carry):
        base = j * (nsub * _GRP)
        dtg = dt_s[pl.ds(base, nsub * _GRP), :]
        xcg = xc_s[pl.ds(base, nsub * _GRP), :]
        bcg = pltpu.roll(bct_all, -base, axis=1)[:, :nsub * _GRP]
        h = h_s[...]
        for s in range(nsub):
            sl = slice(s * _GRP, (s + 1) * _GRP)
            h = subgroup(base + s * _GRP, dtg[sl], xcg[sl],
                         bcg[:, sl], h)
        h_s[...] = h
        return carry

    jax.lax.fori_loop(0, tc // (nsub * _GRP), body, 0)

    z = z_s[...]
    y_ref[0, 0] = ((y_ref[0, 0] + xc_s[...] * dp_ref[0])
                   * (z * jax.nn.sigmoid(z)))


def _merge_kernel(yf_ref, yb_ref, j_ref, wf_ref, wb_ref, x_ref, mb_ref, o_ref):
    ub = jnp.dot(yb_ref[0, 0], wb_ref[...], preferred_element_type=jnp.float32)
    o_ref[0] = (jnp.dot(yf_ref[0, 0], wf_ref[...],
                        preferred_element_type=jnp.float32)
                + jnp.dot(j_ref[...], ub, preferred_element_type=jnp.float32)
                + x_ref[0] + mb_ref[...])


def kernel(x, norm_g, norm_b, p_fwd, p_bwd, merge_w, merge_b):
    B, L, dm = x.shape
    di = D_INNER_
    ds = D_STATE_

    def prep(p):
        in_w, conv_w, conv_b, xproj_w, dt_w, dt_b, A_log, Dp, out_w = p
        return dict(
            winT=in_w.T,
            convw=conv_w.T,
            convb=conv_b[None, :],
            xpdtT=xproj_w[:DT_RANK_].T,
            bcw=xproj_w[DT_RANK_:],
            dtwT=dt_w.T,
            dtb=dt_b[None, :],
            aT=(-jnp.exp(A_log)).T,
            dp=Dp[None, :],
            out_w=out_w,
        )

    pf, pb = prep(p_fwd), prep(p_bwd)

    def stk(k):
        return jnp.stack([pf[k], pb[k]])

    winT = stk('winT')
    convw = stk('convw')
    convb = stk('convb')
    xpdtT = stk('xpdtT')
    bcw = stk('bcw')
    dtwT = stk('dtwT')
    dtb = stk('dtb')
    aT = stk('aT')
    dp = stk('dp')

    hp = jax.lax.Precision.HIGHEST
    wfT = jnp.dot(pf['out_w'].T, merge_w[:, :dm].T, precision=hp)
    wbT = jnp.dot(pb['out_w'].T, merge_w[:, dm:].T, precision=hp)

    nc = L // _TC
    nbr = L // 8
    jrevt = jnp.flipud(jnp.eye(_TC, dtype=jnp.float32))
    jrev8 = jnp.flipud(jnp.eye(8, dtype=jnp.float32))
    y = pl.pallas_call(
        _scan_kernel,
        grid=(2, B, nc),
        in_specs=[
            pl.BlockSpec((1, _TC, dm),
                         lambda d, b, c: (b, c + d * (nc - 1 - 2 * c), 0)),
            pl.BlockSpec((1, 8, dm),
                         lambda d, b, c: (b, jnp.where(
                             d == 0,
                             jnp.maximum(c * (_TC // 8) - 1, 0),
                             jnp.minimum((nc - c) * (_TC // 8), nbr - 1)), 0)),
            pl.BlockSpec((_TC, _TC), lambda d, b, c: (0, 0)),
            pl.BlockSpec((8, 8), lambda d, b, c: (0, 0)),
            pl.BlockSpec((1, dm), lambda d, b, c: (0, 0)),
            pl.BlockSpec((1, dm), lambda d, b, c: (0, 0)),
            pl.BlockSpec((1, dm, 2 * di), lambda d, b, c: (d, 0, 0)),
            pl.BlockSpec((1, 4, di), lambda d, b, c: (d, 0, 0)),
            pl.BlockSpec((1, 1, di), lambda d, b, c: (d, 0, 0)),
            pl.BlockSpec((1, di, DT_RANK_), lambda d, b, c: (d, 0, 0)),
            pl.BlockSpec((1, DT_RANK_, di), lambda d, b, c: (d, 0, 0)),
            pl.BlockSpec((1, 1, di), lambda d, b, c: (d, 0, 0)),
            pl.BlockSpec((1, 2 * ds, di), lambda d, b, c: (d, 0, 0)),
            pl.BlockSpec((1, ds, di), lambda d, b, c: (d, 0, 0)),
            pl.BlockSpec((1, 1, di), lambda d, b, c: (d, 0, 0)),
        ],
        out_specs=pl.BlockSpec((1, 1, _TC, di), lambda d, b, c: (d, b, c, 0)),
        out_shape=jax.ShapeDtypeStruct((2, B, L, di), jnp.float32),
        scratch_shapes=[
            pltpu.VMEM((_TC, di), jnp.float32),
            pltpu.VMEM((_TC, di), jnp.float32),
            pltpu.VMEM((_TC, di), jnp.float32),
            pltpu.VMEM((2 * ds, _TC), jnp.float32),
            pltpu.VMEM((ds, di), jnp.float32),
        ],
        compiler_params=pltpu.CompilerParams(
            dimension_semantics=("parallel", "parallel", "arbitrary"),
            vmem_limit_bytes=50 * 1024 * 1024),
        name="mamba_scan",
    )(x, x, jrevt, jrev8, norm_g[None, :], norm_b[None, :], winT,
      convw, convb, xpdtT, dtwT, dtb, bcw, aT, dp)

    nrt3 = L // _RT3
    jrev3 = jnp.flipud(jnp.eye(_RT3, dtype=jnp.float32))
    out = pl.pallas_call(
        _merge_kernel,
        grid=(B, nrt3),
        in_specs=[
            pl.BlockSpec((1, 1, _RT3, di), lambda b, r: (0, b, r, 0)),
            pl.BlockSpec((1, 1, _RT3, di), lambda b, r: (1, b, nrt3 - 1 - r, 0)),
            pl.BlockSpec((_RT3, _RT3), lambda b, r: (0, 0)),
            pl.BlockSpec((di, dm), lambda b, r: (0, 0)),
            pl.BlockSpec((di, dm), lambda b, r: (0, 0)),
            pl.BlockSpec((1, _RT3, dm), lambda b, r: (b, r, 0)),
            pl.BlockSpec((1, dm), lambda b, r: (0, 0)),
        ],
        out_specs=pl.BlockSpec((1, _RT3, dm), lambda b, r: (b, r, 0)),
        out_shape=jax.ShapeDtypeStruct((B, L, dm), jnp.float32),
        compiler_params=pltpu.CompilerParams(
            dimension_semantics=("parallel", "parallel")),
        name="merge",
    )(y, y, jrev3, wfT, wbT, x, merge_b[None, :])

    return out


# speedup vs baseline: 1.1614x; 1.0089x over previous
import jax
import jax.numpy as jnp
from jax.experimental import pallas as pl
from jax.experimental.pallas import tpu as pltpu

D_MODEL_ = 512
D_STATE_ = 16
D_CONV_ = 4
D_INNER_ = 1024
DT_RANK_ = 32

_TC = 256
_GRP = 8
_RT3 = 512


def _scan_kernel(x_ref, xh_ref, jt_ref, j8_ref, g_ref, b_ref, win_ref,
                 convw_ref, convb_ref, xpdt_ref, dtw_ref,
                 dtb_ref, bcw_ref, at_ref, dp_ref, y_ref,
                 z_s, xc_s, dt_s, bct_s, h_s):
    d = pl.program_id(0)
    c = pl.program_id(2)
    tc = xc_s.shape[0]

    def ln(t):
        mu = jnp.mean(t, axis=1, keepdims=True)
        v = t - mu
        var = jnp.mean(v * v, axis=1, keepdims=True)
        return v * jax.lax.rsqrt(var + 1e-5) * g_ref[...] + b_ref[...]

    xn = ln(x_ref[0])
    xn = jnp.where(d == 0, xn,
                   jnp.dot(jt_ref[...], xn, preferred_element_type=jnp.float32))
    xnh = ln(xh_ref[0])
    xnh = jnp.where(d == 0, xnh,
                    jnp.dot(j8_ref[...], xnh,
                            preferred_element_type=jnp.float32))
    xzc = jnp.dot(xn, win_ref[0], preferred_element_type=jnp.float32)
    xi = xzc[:, :D_INNER_]
    z_s[...] = xzc[:, D_INNER_:]
    xih = jnp.dot(xnh, win_ref[0], preferred_element_type=jnp.float32)

    prev3 = xih[5:8, :D_INNER_]
    prev3 = jnp.where(c == 0, 0.0, prev3)
    xp = jnp.concatenate([prev3, xi], axis=0)
    w = convw_ref[0]
    xc = (xp[0:tc] * w[0:1, :] + xp[1:tc + 1] * w[1:2, :]
          + xp[2:tc + 2] * w[2:3, :] + xp[3:tc + 3] * w[3:4, :]
          + convb_ref[0])
    xc = xc * jax.nn.sigmoid(xc)
    xc_s[...] = xc

    dtr = jnp.dot(xc, xpdt_ref[0], preferred_element_type=jnp.float32)
    dt_lin = jnp.dot(dtr, dtw_ref[0],
                     preferred_element_type=jnp.float32) + dtb_ref[0]
    dt_s[...] = jax.nn.softplus(dt_lin)
    bct_s[...] = jax.lax.dot_general(
        bcw_ref[0], xc, (((1,), (1,)), ((), ())),
        preferred_element_type=jnp.float32)

    aT = at_ref[0]

    @pl.when(c == 0)
    def _():
        h_s[...] = jnp.zeros_like(h_s)

    bct_all = bct_s[...]

    def subgroup(base, dt8, xc8, bc8, h):
        dA8 = jnp.exp(dt8[:, None, :] * aT[None])
        bx8 = (dt8 * xc8)[:, None, :]
        ys = []
        for r in range(_GRP):
            bcol = bc8[0:D_STATE_, r:r + 1]
            ccol = bc8[D_STATE_:2 * D_STATE_, r:r + 1]
            h = dA8[r] * h + bx8[r] * bcol
            ys.append(jnp.sum(ccol * h, axis=0, keepdims=True))
        y_ref[0, 0, pl.ds(base, _GRP), :] = jnp.concatenate(ys, axis=0)
        return h

    nsub = 16
    def body(j, carry):
        base = j * (nsub * _GRP)
        dtg = dt_s[pl.ds(base, nsub * _GRP), :]
        xcg = xc_s[pl.ds(base, nsub * _GRP), :]
        bcg = pltpu.roll(bct_all, -base, axis=1)[:, :nsub * _GRP]
        h = h_s[...]
        for s in range(nsub):
            sl = slice(s * _GRP, (s + 1) * _GRP)
            h = subgroup(base + s * _GRP, dtg[sl], xcg[sl],
                         bcg[:, sl], h)
        h_s[...] = h
        return carry

    jax.lax.fori_loop(0, tc // (nsub * _GRP), body, 0)

    z = z_s[...]
    y_ref[0, 0] = ((y_ref[0, 0] + xc_s[...] * dp_ref[0])
                   * (z * jax.nn.sigmoid(z)))


def _merge_kernel(yf_ref, yb_ref, j_ref, wf_ref, wb_ref, x_ref, mb_ref, o_ref):
    ub = jnp.dot(yb_ref[0, 0], wb_ref[...], preferred_element_type=jnp.float32)
    o_ref[0] = (jnp.dot(yf_ref[0, 0], wf_ref[...],
                        preferred_element_type=jnp.float32)
                + jnp.dot(j_ref[...], ub, preferred_element_type=jnp.float32)
                + x_ref[0] + mb_ref[...])


def kernel(x, norm_g, norm_b, p_fwd, p_bwd, merge_w, merge_b):
    B, L, dm = x.shape
    di = D_INNER_
    ds = D_STATE_

    def prep(p):
        in_w, conv_w, conv_b, xproj_w, dt_w, dt_b, A_log, Dp, out_w = p
        return dict(
            winT=in_w.T,
            convw=conv_w.T,
            convb=conv_b[None, :],
            xpdtT=xproj_w[:DT_RANK_].T,
            bcw=xproj_w[DT_RANK_:],
            dtwT=dt_w.T,
            dtb=dt_b[None, :],
            aT=(-jnp.exp(A_log)).T,
            dp=Dp[None, :],
            out_w=out_w,
        )

    pf, pb = prep(p_fwd), prep(p_bwd)

    def stk(k):
        return jnp.stack([pf[k], pb[k]])

    winT = stk('winT')
    convw = stk('convw')
    convb = stk('convb')
    xpdtT = stk('xpdtT')
    bcw = stk('bcw')
    dtwT = stk('dtwT')
    dtb = stk('dtb')
    aT = stk('aT')
    dp = stk('dp')

    hp = jax.lax.Precision.HIGHEST
    wfT = jnp.dot(pf['out_w'].T, merge_w[:, :dm].T, precision=hp)
    wbT = jnp.dot(pb['out_w'].T, merge_w[:, dm:].T, precision=hp)

    nc = L // _TC
    nbr = L // 8
    jrevt = jnp.flipud(jnp.eye(_TC, dtype=jnp.float32))
    jrev8 = jnp.flipud(jnp.eye(8, dtype=jnp.float32))
    y = pl.pallas_call(
        _scan_kernel,
        grid=(2, B, nc),
        in_specs=[
            pl.BlockSpec((1, _TC, dm),
                         lambda d, b, c: (b, c + d * (nc - 1 - 2 * c), 0)),
            pl.BlockSpec((1, 8, dm),
                         lambda d, b, c: (b, jnp.where(
                             d == 0,
                             jnp.maximum(c * (_TC // 8) - 1, 0),
                             jnp.minimum((nc - c) * (_TC // 8), nbr - 1)), 0)),
            pl.BlockSpec((_TC, _TC), lambda d, b, c: (0, 0)),
            pl.BlockSpec((8, 8), lambda d, b, c: (0, 0)),
            pl.BlockSpec((1, dm), lambda d, b, c: (0, 0)),
            pl.BlockSpec((1, dm), lambda d, b, c: (0, 0)),
            pl.BlockSpec((1, dm, 2 * di), lambda d, b, c: (d, 0, 0)),
            pl.BlockSpec((1, 4, di), lambda d, b, c: (d, 0, 0)),
            pl.BlockSpec((1, 1, di), lambda d, b, c: (d, 0, 0)),
            pl.BlockSpec((1, di, DT_RANK_), lambda d, b, c: (d, 0, 0)),
            pl.BlockSpec((1, DT_RANK_, di), lambda d, b, c: (d, 0, 0)),
            pl.BlockSpec((1, 1, di), lambda d, b, c: (d, 0, 0)),
            pl.BlockSpec((1, 2 * ds, di), lambda d, b, c: (d, 0, 0)),
            pl.BlockSpec((1, ds, di), lambda d, b, c: (d, 0, 0)),
            pl.BlockSpec((1, 1, di), lambda d, b, c: (d, 0, 0)),
        ],
        out_specs=pl.BlockSpec((1, 1, _TC, di), lambda d, b, c: (d, b, c, 0)),
        out_shape=jax.ShapeDtypeStruct((2, B, L, di), jnp.float32),
        scratch_shapes=[
            pltpu.VMEM((_TC, di), jnp.float32),
            pltpu.VMEM((_TC, di), jnp.float32),
            pltpu.VMEM((_TC, di), jnp.float32),
            pltpu.VMEM((2 * ds, _TC), jnp.float32),
            pltpu.VMEM((ds, di), jnp.float32),
        ],
        compiler_params=pltpu.CompilerParams(
            dimension_semantics=("parallel", "parallel", "arbitrary"),
            vmem_limit_bytes=50 * 1024 * 1024),
        name="mamba_scan",
    )(x, x, jrevt, jrev8, norm_g[None, :], norm_b[None, :], winT,
      convw, convb, xpdtT, dtwT, dtb, bcw, aT, dp)

    nrt3 = L // _RT3
    jrev3 = jnp.flipud(jnp.eye(_RT3, dtype=jnp.float32))
    out = pl.pallas_call(
        _merge_kernel,
        grid=(B, nrt3),
        in_specs=[
            pl.BlockSpec((1, 1, _RT3, di), lambda b, r: (0, b, r, 0)),
            pl.BlockSpec((1, 1, _RT3, di), lambda b, r: (1, b, nrt3 - 1 - r, 0)),
            pl.BlockSpec((_RT3, _RT3), lambda b, r: (0, 0)),
            pl.BlockSpec((di, dm), lambda b, r: (0, 0)),
            pl.BlockSpec((di, dm), lambda b, r: (0, 0)),
            pl.BlockSpec((1, _RT3, dm), lambda b, r: (b, r, 0)),
            pl.BlockSpec((1, dm), lambda b, r: (0, 0)),
        ],
        out_specs=pl.BlockSpec((1, _RT3, dm), lambda b, r: (b, r, 0)),
        out_shape=jax.ShapeDtypeStruct((B, L, dm), jnp.float32),
        compiler_params=pltpu.CompilerParams(
            dimension_semantics=("parallel", "parallel")),
        name="merge",
    )(y, y, jrev3, wfT, wbT, x, merge_b[None, :])

    return out
